# baseline (speedup 1.0000x reference)
import functools

import jax
import jax.numpy as jnp
from jax import lax
from jax.experimental import pallas as pl
from jax.experimental.pallas import tpu as pltpu
from jax.experimental.pallas import tpu_sc as plsc

_D = 128
_NC, _NS = 2, 16
_NW = _NC * _NS
_CHUNK = 128
_NBUF = 5
_BLK = 4096
_VPAD = 384
_SC_FRAC_NUM, _SC_FRAC_DEN = 1, 2


def _sc_add_pe(xf, pos, pe, n_sc):
    rows_per_tile = n_sc // _NW
    n_chunks = rows_per_tile // _CHUNK

    mesh = plsc.VectorSubcoreMesh(
        core_axis_name="c", subcore_axis_name="s",
        num_cores=_NC, num_subcores=_NS)

    @functools.partial(
        pl.kernel,
        out_type=jax.ShapeDtypeStruct((n_sc, _D), jnp.float32),
        mesh=mesh,
        compiler_params=pltpu.CompilerParams(needs_layout_passes=False),
        scratch_types=[
            [pltpu.VMEM((_CHUNK, _D), jnp.float32) for _ in range(_NBUF)],
            [pltpu.VMEM((_CHUNK,), jnp.int32) for _ in range(_NBUF)],
            [pltpu.SemaphoreType.DMA for _ in range(_NBUF)],
            [pltpu.SemaphoreType.DMA for _ in range(_NBUF)],
            [pltpu.SemaphoreType.DMA for _ in range(_NBUF)],
            [pltpu.SemaphoreType.DMA for _ in range(_NBUF)],
            pltpu.VMEM_SHARED((365, _D), jnp.float32),
            pltpu.SemaphoreType.DMA,
        ],
    )
    def k(x_hbm, pos_hbm, pe_hbm, out_hbm,
          bufs, poss, isems, psems, gsems, osems, pe_sh, pe_sem):
        wid = lax.axis_index("s") * _NC + lax.axis_index("c")
        base = wid * rows_per_tile

        @pl.when(lax.axis_index("s") == 0)
        def _():
            pltpu.async_copy(pe_hbm, pe_sh, pe_sem).wait()

        plsc.subcore_barrier()

        def start_in(c, b):
            r0 = base + c * _CHUNK
            pltpu.make_async_copy(
                x_hbm.at[pl.ds(r0, _CHUNK)], bufs[b], isems[b]).start()
            pltpu.make_async_copy(
                pos_hbm.at[pl.ds(r0, _CHUNK)], poss[b], psems[b]).start()

        def wait_in(b):
            pltpu.make_async_copy(
                x_hbm.at[pl.ds(base, _CHUNK)], bufs[b], isems[b]).wait()
            pltpu.make_async_copy(
                pos_hbm.at[pl.ds(base, _CHUNK)], poss[b], psems[b]).wait()

        def start_gather(b):
            pltpu.async_copy(pe_sh.at[poss[b]], bufs[b], gsems[b], add=True)

        def wait_gather(b):
            pltpu.make_async_copy(pe_sh.at[poss[b]], bufs[b], gsems[b]).wait()

        def wait_out(b):
            pltpu.make_async_copy(
                bufs[b], out_hbm.at[pl.ds(base, _CHUNK)], osems[b]).wait()

        for b in range(_NBUF):
            start_in(b, b)
        wait_in(0)
        start_gather(0)

        def body(c5, carry):
            for b in range(_NBUF):
                c = c5 * _NBUF + b
                bn = (b + 1) % _NBUF

                @pl.when(c + 1 < n_chunks)
                def _():
                    wait_in(bn)
                    start_gather(bn)

                wait_gather(b)
                pltpu.make_async_copy(
                    bufs[b],
                    out_hbm.at[pl.ds(base + c * _CHUNK, _CHUNK)],
                    osems[b]).start()

                br = (b + _NBUF - 1) % _NBUF

                @pl.when((c >= 1) & (c + _NBUF - 1 < n_chunks))
                def _():
                    wait_out(br)
                    start_in(c + _NBUF - 1, br)
            return carry

        lax.fori_loop(0, n_chunks // _NBUF, body, 0)
        for b in range(_NBUF):
            wait_out(b)

    return k(xf, pos, pe)


def _tc_body(pos_ref, pe_ref, x_ref, o_ref):
    pos = pos_ref[0, 0, :]
    oh = (pos[:, None] == lax.broadcasted_iota(
        jnp.int32, (_BLK, _VPAD), 1)).astype(jnp.bfloat16)
    enc = lax.dot_general(
        oh, pe_ref[...],
        dimension_numbers=(((1,), (0,)), ((), ())),
        preferred_element_type=jnp.float32)
    o_ref[...] = x_ref[...] + enc


def _tc_add_pe(xf, posf, pe, n_sc):
    n = posf.shape[0]
    n_tc = n - n_sc
    blk0 = n_sc // _BLK
    posr = posf.reshape(1, n // _BLK, _BLK).swapaxes(0, 1)
    pe_pad = jnp.zeros((_VPAD, _D), jnp.bfloat16).at[:365].set(
        pe.astype(jnp.bfloat16))
    return pl.pallas_call(
        _tc_body,
        grid=(n_tc // _BLK,),
        in_specs=[
            pl.BlockSpec((1, 1, _BLK), lambda i: (i + blk0, 0, 0)),
            pl.BlockSpec((_VPAD, _D), lambda i: (0, 0)),
            pl.BlockSpec((_BLK, _D), lambda i: (i + blk0, 0)),
        ],
        out_specs=pl.BlockSpec((_BLK, _D), lambda i: (i, 0)),
        out_shape=jax.ShapeDtypeStruct((n_tc, _D), jnp.float32),
        compiler_params=pltpu.CompilerParams(
            dimension_semantics=("arbitrary",)),
    )(posr, pe_pad, xf)


def kernel(x, positions, pe):
    b, s, d = x.shape
    n = b * s
    n_sc = (n * _SC_FRAC_NUM // _SC_FRAC_DEN) // (_NW * _CHUNK * _NBUF) \
        * (_NW * _CHUNK * _NBUF)
    xf = x.reshape(n, d)
    posf = positions.reshape(n)
    out_sc = _sc_add_pe(xf, posf, pe, n_sc)
    out_tc = _tc_add_pe(xf, posf, pe, n_sc)
    return jnp.concatenate([out_sc, out_tc], axis=0).reshape(b, s, d)


# speedup vs baseline: 1.6158x; 1.6158x over previous
import functools

import jax
import jax.numpy as jnp
from jax import lax
from jax.experimental import pallas as pl
from jax.experimental.pallas import tpu as pltpu
from jax.experimental.pallas import tpu_sc as plsc

_D = 128
_NC, _NS = 2, 16
_NW = _NC * _NS
_CHUNK = 256
_G = 128
_NBUF = 3


def _sc_add_pe(xf, pos, pe):
    n = pos.shape[0]
    rows_per_tile = n // _NW
    n_chunks = rows_per_tile // _CHUNK

    mesh = plsc.VectorSubcoreMesh(
        core_axis_name="c", subcore_axis_name="s",
        num_cores=_NC, num_subcores=_NS)

    @functools.partial(
        pl.kernel,
        out_type=jax.ShapeDtypeStruct((n, _D), jnp.float32),
        mesh=mesh,
        compiler_params=pltpu.CompilerParams(needs_layout_passes=False),
        scratch_types=[
            [pltpu.VMEM((_CHUNK, _D), jnp.float32) for _ in range(_NBUF)],
            [pltpu.VMEM((_CHUNK,), jnp.int32) for _ in range(_NBUF)],
            [pltpu.SemaphoreType.DMA for _ in range(_NBUF)],
            [pltpu.SemaphoreType.DMA for _ in range(_NBUF)],
            [pltpu.SemaphoreType.DMA for _ in range(_NBUF)],
            [pltpu.SemaphoreType.DMA for _ in range(_NBUF)],
            pltpu.VMEM_SHARED((365, _D), jnp.float32),
            pltpu.SemaphoreType.DMA,
        ],
    )
    def k(x_hbm, pos_hbm, pe_hbm, out_hbm,
          bufs, poss, isems, psems, gsems, osems, pe_sh, pe_sem):
        wid = lax.axis_index("s") * _NC + lax.axis_index("c")
        base = wid * rows_per_tile

        @pl.when(lax.axis_index("s") == 0)
        def _():
            pltpu.async_copy(pe_hbm, pe_sh, pe_sem).wait()

        plsc.subcore_barrier()

        def start_in(c, b):
            r0 = base + c * _CHUNK
            pltpu.make_async_copy(
                x_hbm.at[pl.ds(r0, _CHUNK)], bufs[b], isems[b]).start()
            pltpu.make_async_copy(
                pos_hbm.at[pl.ds(r0, _CHUNK)], poss[b], psems[b]).start()

        def wait_in(b):
            pltpu.make_async_copy(
                x_hbm.at[pl.ds(base, _CHUNK)], bufs[b], isems[b]).wait()
            pltpu.make_async_copy(
                pos_hbm.at[pl.ds(base, _CHUNK)], poss[b], psems[b]).wait()

        def start_gather(b):
            for h in range(_CHUNK // _G):
                pltpu.async_copy(
                    pe_sh.at[poss[b].at[pl.ds(h * _G, _G)]],
                    bufs[b].at[pl.ds(h * _G, _G)],
                    gsems[b], add=True)

        def wait_gather(b):
            for h in range(_CHUNK // _G):
                pltpu.make_async_copy(
                    pe_sh.at[poss[b].at[pl.ds(h * _G, _G)]],
                    bufs[b].at[pl.ds(h * _G, _G)],
                    gsems[b]).wait()

        def wait_out(b):
            pltpu.make_async_copy(
                bufs[b], out_hbm.at[pl.ds(base, _CHUNK)], osems[b]).wait()

        for b in range(_NBUF):
            start_in(b, b)
        wait_in(0)
        start_gather(0)

        def body(c3, carry):
            for b in range(_NBUF):
                c = c3 * _NBUF + b
                bn = (b + 1) % _NBUF

                @pl.when(c + 1 < n_chunks)
                def _():
                    wait_in(bn)
                    start_gather(bn)

                wait_gather(b)
                pltpu.make_async_copy(
                    bufs[b],
                    out_hbm.at[pl.ds(base + c * _CHUNK, _CHUNK)],
                    osems[b]).start()

                br = (b + _NBUF - 1) % _NBUF

                @pl.when((c >= 1) & (c + _NBUF - 1 < n_chunks))
                def _():
                    wait_out(br)
                    start_in(c + _NBUF - 1, br)
            return carry

        n_trips = n_chunks // _NBUF
        lax.fori_loop(0, n_trips, body, 0)
        for c in range(n_trips * _NBUF, n_chunks):
            b = c % _NBUF
            wait_gather(b)
            pltpu.make_async_copy(
                bufs[b],
                out_hbm.at[pl.ds(base + c * _CHUNK, _CHUNK)],
                osems[b]).start()
        for b in range(_NBUF):
            wait_out(b)

    return k(xf, pos, pe)


def kernel(x, positions, pe):
    b, s, d = x.shape
    out = _sc_add_pe(x.reshape(b * s, d), positions.reshape(b * s), pe)
    return out.reshape(b, s, d)


# speedup vs baseline: 1.6795x; 1.0394x over previous
import functools

import jax
import jax.numpy as jnp
from jax import lax
from jax.experimental import pallas as pl
from jax.experimental.pallas import tpu as pltpu
from jax.experimental.pallas import tpu_sc as plsc

_D = 128
_NC, _NS = 2, 16
_NW = _NC * _NS
_CHUNK = 128
_NBUF = 6


def _sc_add_pe(xf, pos, pe):
    n = pos.shape[0]
    rows_per_tile = n // _NW
    n_chunks = rows_per_tile // _CHUNK

    mesh = plsc.VectorSubcoreMesh(
        core_axis_name="c", subcore_axis_name="s",
        num_cores=_NC, num_subcores=_NS)

    @functools.partial(
        pl.kernel,
        out_type=jax.ShapeDtypeStruct((n, _D), jnp.float32),
        mesh=mesh,
        compiler_params=pltpu.CompilerParams(needs_layout_passes=False),
        scratch_types=[
            [pltpu.VMEM((_CHUNK, _D), jnp.float32) for _ in range(_NBUF)],
            [pltpu.VMEM((_CHUNK,), jnp.int32) for _ in range(_NBUF)],
            [pltpu.SemaphoreType.DMA for _ in range(_NBUF)],
            [pltpu.SemaphoreType.DMA for _ in range(_NBUF)],
            [pltpu.SemaphoreType.DMA for _ in range(_NBUF)],
            [pltpu.SemaphoreType.DMA for _ in range(_NBUF)],
            pltpu.VMEM_SHARED((365, _D), jnp.float32),
            pltpu.SemaphoreType.DMA,
        ],
    )
    def k(x_hbm, pos_hbm, pe_hbm, out_hbm,
          bufs, poss, isems, psems, gsems, osems, pe_sh, pe_sem):
        wid = lax.axis_index("s") * _NC + lax.axis_index("c")
        base = wid * rows_per_tile

        @pl.when(lax.axis_index("s") == 0)
        def _():
            pltpu.async_copy(pe_hbm, pe_sh, pe_sem).wait()

        plsc.subcore_barrier()

        def start_in(c, b):
            r0 = base + c * _CHUNK
            pltpu.make_async_copy(
                x_hbm.at[pl.ds(r0, _CHUNK)], bufs[b], isems[b]).start()
            pltpu.make_async_copy(
                pos_hbm.at[pl.ds(r0, _CHUNK)], poss[b], psems[b]).start()

        def wait_in(b):
            pltpu.make_async_copy(
                x_hbm.at[pl.ds(base, _CHUNK)], bufs[b], isems[b]).wait()
            pltpu.make_async_copy(
                pos_hbm.at[pl.ds(base, _CHUNK)], poss[b], psems[b]).wait()

        def start_gather(b):
            pltpu.async_copy(pe_sh.at[poss[b]], bufs[b], gsems[b], add=True)

        def wait_gather(b):
            pltpu.make_async_copy(pe_sh.at[poss[b]], bufs[b], gsems[b]).wait()

        def wait_out(b):
            pltpu.make_async_copy(
                bufs[b], out_hbm.at[pl.ds(base, _CHUNK)], osems[b]).wait()

        for b in range(_NBUF):
            start_in(b, b)
        wait_in(0)
        start_gather(0)

        def body(c5, carry):
            for b in range(_NBUF):
                c = c5 * _NBUF + b
                bn = (b + 1) % _NBUF

                @pl.when(c + 1 < n_chunks)
                def _():
                    wait_in(bn)
                    start_gather(bn)

                wait_gather(b)
                pltpu.make_async_copy(
                    bufs[b],
                    out_hbm.at[pl.ds(base + c * _CHUNK, _CHUNK)],
                    osems[b]).start()

                br = (b + _NBUF - 1) % _NBUF

                @pl.when((c >= 1) & (c + _NBUF - 1 < n_chunks))
                def _():
                    wait_out(br)
                    start_in(c + _NBUF - 1, br)
            return carry

        n_trips = n_chunks // _NBUF
        lax.fori_loop(0, n_trips, body, 0)
        for c in range(n_trips * _NBUF, n_chunks):
            b = c % _NBUF
            if c + 1 < n_chunks:
                wait_in((b + 1) % _NBUF)
                start_gather((b + 1) % _NBUF)
            wait_gather(b)
            pltpu.make_async_copy(
                bufs[b],
                out_hbm.at[pl.ds(base + c * _CHUNK, _CHUNK)],
                osems[b]).start()
        for b in range(_NBUF):
            wait_out(b)

    return k(xf, pos, pe)


def kernel(x, positions, pe):
    b, s, d = x.shape
    out = _sc_add_pe(x.reshape(b * s, d), positions.reshape(b * s), pe)
    return out.reshape(b, s, d)


# speedup vs baseline: 1.6826x; 1.0018x over previous
import functools

import jax
import jax.numpy as jnp
from jax import lax
from jax.experimental import pallas as pl
from jax.experimental.pallas import tpu as pltpu
from jax.experimental.pallas import tpu_sc as plsc

_D = 128
_NC, _NS = 2, 16
_NW = _NC * _NS
_CHUNK = 128
_NBUF = 6


def _sc_add_pe(xf, pos, pe):
    n = pos.shape[0]
    rows_per_tile = n // _NW
    n_chunks = rows_per_tile // _CHUNK

    mesh = plsc.VectorSubcoreMesh(
        core_axis_name="c", subcore_axis_name="s",
        num_cores=_NC, num_subcores=_NS)

    @functools.partial(
        pl.kernel,
        out_type=jax.ShapeDtypeStruct((n, _D), jnp.float32),
        mesh=mesh,
        compiler_params=pltpu.CompilerParams(needs_layout_passes=False),
        scratch_types=[
            [pltpu.VMEM((_CHUNK, _D), jnp.float32) for _ in range(_NBUF)],
            [pltpu.VMEM((_CHUNK,), jnp.int32) for _ in range(_NBUF)],
            [pltpu.SemaphoreType.DMA for _ in range(_NBUF)],
            [pltpu.SemaphoreType.DMA for _ in range(_NBUF)],
            [pltpu.SemaphoreType.DMA for _ in range(_NBUF)],
            [pltpu.SemaphoreType.DMA for _ in range(_NBUF)],
            pltpu.VMEM_SHARED((365, _D), jnp.float32),
            pltpu.SemaphoreType.DMA,
        ],
    )
    def k(x_hbm, pos_hbm, pe_hbm, out_hbm,
          bufs, poss, isems, psems, gsems, osems, pe_sh, pe_sem):
        wid = lax.axis_index("s") * _NC + lax.axis_index("c")
        base = wid * rows_per_tile

        @pl.when(lax.axis_index("s") == 0)
        def _():
            pltpu.async_copy(pe_hbm, pe_sh, pe_sem).wait()

        plsc.subcore_barrier()

        def start_in(c, b):
            r0 = base + c * _CHUNK
            pltpu.make_async_copy(
                x_hbm.at[pl.ds(r0, _CHUNK)], bufs[b], isems[b]).start()
            pltpu.make_async_copy(
                pos_hbm.at[pl.ds(r0, _CHUNK)], poss[b], psems[b]).start()

        def wait_in(b):
            pltpu.make_async_copy(
                x_hbm.at[pl.ds(base, _CHUNK)], bufs[b], isems[b]).wait()
            pltpu.make_async_copy(
                pos_hbm.at[pl.ds(base, _CHUNK)], poss[b], psems[b]).wait()

        def start_gather(b):
            pltpu.async_copy(pe_sh.at[poss[b]], bufs[b], gsems[b], add=True)

        def wait_gather(b):
            pltpu.make_async_copy(pe_sh.at[poss[b]], bufs[b], gsems[b]).wait()

        def wait_out(b):
            pltpu.make_async_copy(
                bufs[b], out_hbm.at[pl.ds(base, _CHUNK)], osems[b]).wait()

        for b in range(_NBUF):
            start_in(b, b)
        for c in range(2):
            wait_in(c)
            start_gather(c)

        def body(c5, carry):
            for b in range(_NBUF):
                c = c5 * _NBUF + b
                bn = (b + 2) % _NBUF

                @pl.when(c + 2 < n_chunks)
                def _():
                    wait_in(bn)
                    start_gather(bn)

                wait_gather(b)
                pltpu.make_async_copy(
                    bufs[b],
                    out_hbm.at[pl.ds(base + c * _CHUNK, _CHUNK)],
                    osems[b]).start()

                br = (b + _NBUF - 1) % _NBUF

                @pl.when((c >= 1) & (c + _NBUF - 1 < n_chunks))
                def _():
                    wait_out(br)
                    start_in(c + _NBUF - 1, br)
            return carry

        n_trips = n_chunks // _NBUF
        lax.fori_loop(0, n_trips, body, 0)
        for c in range(n_trips * _NBUF, n_chunks):
            b = c % _NBUF
            wait_gather(b)
            pltpu.make_async_copy(
                bufs[b],
                out_hbm.at[pl.ds(base + c * _CHUNK, _CHUNK)],
                osems[b]).start()
        for b in range(_NBUF):
            wait_out(b)

    return k(xf, pos, pe)


def kernel(x, positions, pe):
    b, s, d = x.shape
    out = _sc_add_pe(x.reshape(b * s, d), positions.reshape(b * s), pe)
    return out.reshape(b, s, d)
